# baseline (speedup 1.0000x reference)
import functools

import jax
import jax.numpy as jnp
from jax import lax
from jax.experimental import pallas as pl
from jax.experimental.pallas import tpu as pltpu
from jax.experimental.pallas import tpu_sc as plsc

N = 1600000
NUM_SEGMENTS = 4096
NW = 32
ROWS_PER_W = N // NW
TILE = 2000
STEPS = ROWS_PER_W // TILE
VECS = TILE // 16

_NEG_INF = float("-inf")


def _sc_stage(trunk_flat, votes_flat, batch_idx, wvec):
    mesh = plsc.VectorSubcoreMesh(core_axis_name="c", subcore_axis_name="s")

    @functools.partial(
        pl.kernel,
        mesh=mesh,
        compiler_params=pltpu.CompilerParams(needs_layout_passes=False),
        out_type=jax.ShapeDtypeStruct((NW, NUM_SEGMENTS), jnp.float32),
        scratch_types=[
            pltpu.VMEM((TILE * 3,), jnp.float32),
            pltpu.VMEM((TILE,), jnp.float32),
            pltpu.VMEM((TILE,), jnp.int32),
            pltpu.VMEM((16, NUM_SEGMENTS), jnp.float32),
            pltpu.VMEM((NUM_SEGMENTS,), jnp.float32),
            pltpu.VMEM((80,), jnp.float32),
        ],
    )
    def k(trunk_hbm, votes_hbm, idx_hbm, wv_hbm, out_hbm,
          trunk_v, votes_v, idx_v, acc, red, wv_v):
        wid = lax.axis_index("s") * 2 + lax.axis_index("c")

        lane = lax.iota(jnp.int32, 16)
        lane3 = lane * 3
        ninf = jnp.full((16,), _NEG_INF, jnp.float32)

        pltpu.sync_copy(wv_hbm, wv_v)
        w0 = wv_v[pl.ds(0, 16)]
        w1 = wv_v[pl.ds(16, 16)]
        w2 = wv_v[pl.ds(32, 16)]
        w3 = wv_v[pl.ds(48, 16)]
        w4 = wv_v[pl.ds(64, 16)]

        def init_body(j, _):
            for r in range(16):
                acc[r, pl.ds(j * 16, 16)] = ninf
            return 0
        lax.fori_loop(0, NUM_SEGMENTS // 16, init_body, 0)

        def step_body(s, _):
            pltpu.sync_copy(
                trunk_hbm.at[pl.ds(wid * (ROWS_PER_W * 3) + s * (TILE * 3),
                                   TILE * 3)],
                trunk_v)
            pltpu.sync_copy(
                votes_hbm.at[pl.ds(wid * ROWS_PER_W + s * TILE, TILE)],
                votes_v)
            pltpu.sync_copy(
                idx_hbm.at[pl.ds(wid * ROWS_PER_W + s * TILE, TILE)],
                idx_v)

            def vec_body(v, _):
                b3 = v * 48
                t0 = plsc.load_gather(trunk_v, [b3 + lane3])
                t1 = plsc.load_gather(trunk_v, [b3 + lane3 + 1])
                t2 = plsc.load_gather(trunk_v, [b3 + lane3 + 2])
                vv = votes_v[pl.ds(v * 16, 16)]
                iv = idx_v[pl.ds(v * 16, 16)]
                x = t0 * w0 + t1 * w1 + t2 * w2 + vv * w3 + w4
                g = plsc.load_gather(acc, [lane, iv])
                plsc.store_scatter(acc, [lane, iv], jnp.maximum(g, x))
                return 0
            lax.fori_loop(0, VECS, vec_body, 0)
            return 0
        lax.fori_loop(0, STEPS, step_body, 0)

        def red_body(j, _):
            m = acc[0, pl.ds(j * 16, 16)]
            for r in range(1, 16):
                m = jnp.maximum(m, acc[r, pl.ds(j * 16, 16)])
            red[pl.ds(j * 16, 16)] = m
            return 0
        lax.fori_loop(0, NUM_SEGMENTS // 16, red_body, 0)

        pltpu.sync_copy(red, out_hbm.at[wid])

    return k(trunk_flat, votes_flat, batch_idx, wvec)


def _tc_reduce(partial):
    def body(p_ref, o_ref):
        o_ref[...] = jnp.max(p_ref[...], axis=0, keepdims=True)

    return pl.pallas_call(
        body,
        out_shape=jax.ShapeDtypeStruct((1, NUM_SEGMENTS), jnp.float32),
    )(partial)


def kernel(trunk, votes, batch_idx, W, b):
    trunk_flat = trunk.reshape(-1)
    votes_flat = votes.reshape(-1)
    wcat = jnp.concatenate([W[:, 0], b])
    wvec = jnp.repeat(wcat, 16)
    partial = _sc_stage(trunk_flat, votes_flat, batch_idx, wvec)
    out = _tc_reduce(partial)
    return out.reshape(NUM_SEGMENTS, 1)


# speedup vs baseline: 8.7466x; 8.7466x over previous
import functools

import jax
import jax.numpy as jnp
from jax import lax
from jax.experimental import pallas as pl
from jax.experimental.pallas import tpu as pltpu
from jax.experimental.pallas import tpu_sc as plsc

N = 1600000
NUM_SEGMENTS = 4096
NW = 32
ROWS_PER_W = N // NW
TILE = 2000
STEPS = ROWS_PER_W // TILE
VECS = TILE // 16

_NEG_INF = float("-inf")


def _sc_stage(trunk_flat, votes_flat, batch_idx, wvec):
    mesh = plsc.VectorSubcoreMesh(core_axis_name="c", subcore_axis_name="s")

    @functools.partial(
        pl.kernel,
        mesh=mesh,
        compiler_params=pltpu.CompilerParams(needs_layout_passes=False),
        out_type=jax.ShapeDtypeStruct((NW, NUM_SEGMENTS), jnp.float32),
        scratch_types=[
            pltpu.VMEM((TILE * 3,), jnp.float32),
            pltpu.VMEM((TILE,), jnp.float32),
            pltpu.VMEM((TILE,), jnp.int32),
            pltpu.VMEM((16, NUM_SEGMENTS), jnp.float32),
            pltpu.VMEM((NUM_SEGMENTS,), jnp.float32),
            pltpu.VMEM((80,), jnp.float32),
        ],
    )
    def k(trunk_hbm, votes_hbm, idx_hbm, wv_hbm, out_hbm,
          trunk_v, votes_v, idx_v, acc, red, wv_v):
        wid = lax.axis_index("s") * 2 + lax.axis_index("c")

        lane = lax.iota(jnp.int32, 16)
        ninf = jnp.full((16,), _NEG_INF, jnp.float32)

        pltpu.sync_copy(wv_hbm, wv_v)
        w0 = wv_v[pl.ds(0, 16)]
        w1 = wv_v[pl.ds(16, 16)]
        w2 = wv_v[pl.ds(32, 16)]
        w3 = wv_v[pl.ds(48, 16)]
        w4 = wv_v[pl.ds(64, 16)]

        def init_body(j, _):
            for r in range(16):
                acc[r, pl.ds(j * 16, 16)] = ninf
            return 0
        lax.fori_loop(0, NUM_SEGMENTS // 16, init_body, 0)

        def step_body(s, _):
            row0 = wid * ROWS_PER_W + s * TILE
            for j in range(3):
                pltpu.sync_copy(
                    trunk_hbm.at[pl.ds(j * N + row0, TILE)],
                    trunk_v.at[pl.ds(j * TILE, TILE)])
            pltpu.sync_copy(votes_hbm.at[pl.ds(row0, TILE)], votes_v)
            pltpu.sync_copy(idx_hbm.at[pl.ds(row0, TILE)], idx_v)

            def vec_body(v, _):
                t0 = trunk_v[pl.ds(v * 16, 16)]
                t1 = trunk_v[pl.ds(TILE + v * 16, 16)]
                t2 = trunk_v[pl.ds(2 * TILE + v * 16, 16)]
                vv = votes_v[pl.ds(v * 16, 16)]
                iv = idx_v[pl.ds(v * 16, 16)]
                x = t0 * w0 + t1 * w1 + t2 * w2 + vv * w3 + w4
                g = plsc.load_gather(acc, [lane, iv])
                plsc.store_scatter(acc, [lane, iv], jnp.maximum(g, x))
                return 0
            lax.fori_loop(0, VECS, vec_body, 0)
            return 0
        lax.fori_loop(0, STEPS, step_body, 0)

        def red_body(j, _):
            m = acc[0, pl.ds(j * 16, 16)]
            for r in range(1, 16):
                m = jnp.maximum(m, acc[r, pl.ds(j * 16, 16)])
            red[pl.ds(j * 16, 16)] = m
            return 0
        lax.fori_loop(0, NUM_SEGMENTS // 16, red_body, 0)

        pltpu.sync_copy(red, out_hbm.at[wid])

    return k(trunk_flat, votes_flat, batch_idx, wvec)


def _tc_reduce(partial):
    def body(p_ref, o_ref):
        o_ref[...] = jnp.max(p_ref[...], axis=0, keepdims=True)

    return pl.pallas_call(
        body,
        out_shape=jax.ShapeDtypeStruct((1, NUM_SEGMENTS), jnp.float32),
    )(partial)


def kernel(trunk, votes, batch_idx, W, b):
    trunk_flat = trunk.T.reshape(-1)
    votes_flat = votes.reshape(-1)
    wcat = jnp.concatenate([W[:, 0], b])
    wvec = jnp.repeat(wcat, 16)
    partial = _sc_stage(trunk_flat, votes_flat, batch_idx, wvec)
    out = _tc_reduce(partial)
    return out.reshape(NUM_SEGMENTS, 1)


# speedup vs baseline: 16.1760x; 1.8494x over previous
import functools

import jax
import jax.numpy as jnp
from jax import lax
from jax.experimental import pallas as pl
from jax.experimental.pallas import tpu as pltpu
from jax.experimental.pallas import tpu_sc as plsc

N = 1600000
NUM_SEGMENTS = 4096
NW = 32
ROWS_PER_W = N // NW
TILE = 2000
STEPS = ROWS_PER_W // TILE
VECS = TILE // 16

_NEG_INF = float("-inf")


def _sc_stage(t0, t1, t2, votes_flat, batch_idx, wvec):
    mesh = plsc.VectorSubcoreMesh(core_axis_name="c", subcore_axis_name="s")

    @functools.partial(
        pl.kernel,
        mesh=mesh,
        compiler_params=pltpu.CompilerParams(needs_layout_passes=False),
        out_type=jax.ShapeDtypeStruct((NW, NUM_SEGMENTS), jnp.float32),
        scratch_types=[
            pltpu.VMEM((TILE * 3,), jnp.float32),
            pltpu.VMEM((TILE,), jnp.float32),
            pltpu.VMEM((TILE,), jnp.int32),
            pltpu.VMEM((16, NUM_SEGMENTS), jnp.float32),
            pltpu.VMEM((NUM_SEGMENTS,), jnp.float32),
            pltpu.VMEM((80,), jnp.float32),
        ],
    )
    def k(t0_hbm, t1_hbm, t2_hbm, votes_hbm, idx_hbm, wv_hbm, out_hbm,
          trunk_v, votes_v, idx_v, acc, red, wv_v):
        wid = lax.axis_index("s") * 2 + lax.axis_index("c")

        lane = lax.iota(jnp.int32, 16)
        ninf = jnp.full((16,), _NEG_INF, jnp.float32)

        pltpu.sync_copy(wv_hbm, wv_v)
        w0 = wv_v[pl.ds(0, 16)]
        w1 = wv_v[pl.ds(16, 16)]
        w2 = wv_v[pl.ds(32, 16)]
        w3 = wv_v[pl.ds(48, 16)]
        w4 = wv_v[pl.ds(64, 16)]

        def init_body(j, _):
            for r in range(16):
                acc[r, pl.ds(j * 16, 16)] = ninf
            return 0
        lax.fori_loop(0, NUM_SEGMENTS // 16, init_body, 0)

        def step_body(s, _):
            row0 = wid * ROWS_PER_W + s * TILE
            for j, col in enumerate((t0_hbm, t1_hbm, t2_hbm)):
                pltpu.sync_copy(
                    col.at[pl.ds(row0, TILE)],
                    trunk_v.at[pl.ds(j * TILE, TILE)])
            pltpu.sync_copy(votes_hbm.at[pl.ds(row0, TILE)], votes_v)
            pltpu.sync_copy(idx_hbm.at[pl.ds(row0, TILE)], idx_v)

            def vec_body(v, _):
                t0 = trunk_v[pl.ds(v * 16, 16)]
                t1 = trunk_v[pl.ds(TILE + v * 16, 16)]
                t2 = trunk_v[pl.ds(2 * TILE + v * 16, 16)]
                vv = votes_v[pl.ds(v * 16, 16)]
                iv = idx_v[pl.ds(v * 16, 16)]
                x = t0 * w0 + t1 * w1 + t2 * w2 + vv * w3 + w4
                g = plsc.load_gather(acc, [lane, iv])
                plsc.store_scatter(acc, [lane, iv], jnp.maximum(g, x))
                return 0
            lax.fori_loop(0, VECS, vec_body, 0)
            return 0
        lax.fori_loop(0, STEPS, step_body, 0)

        def red_body(j, _):
            m = acc[0, pl.ds(j * 16, 16)]
            for r in range(1, 16):
                m = jnp.maximum(m, acc[r, pl.ds(j * 16, 16)])
            red[pl.ds(j * 16, 16)] = m
            return 0
        lax.fori_loop(0, NUM_SEGMENTS // 16, red_body, 0)

        pltpu.sync_copy(red, out_hbm.at[wid])

    return k(t0, t1, t2, votes_flat, batch_idx, wvec)


def _tc_reduce(partial):
    def body(p_ref, o_ref):
        o_ref[...] = jnp.max(p_ref[...], axis=0, keepdims=True)

    return pl.pallas_call(
        body,
        out_shape=jax.ShapeDtypeStruct((1, NUM_SEGMENTS), jnp.float32),
    )(partial)


def kernel(trunk, votes, batch_idx, W, b):
    t0, t1, t2 = trunk[:, 0], trunk[:, 1], trunk[:, 2]
    votes_flat = votes.reshape(-1)
    wcat = jnp.concatenate([W[:, 0], b])
    wvec = jnp.repeat(wcat, 16)
    partial = _sc_stage(t0, t1, t2, votes_flat, batch_idx, wvec)
    out = _tc_reduce(partial)
    return out.reshape(NUM_SEGMENTS, 1)


# speedup vs baseline: 21.1578x; 1.3080x over previous
import functools

import jax
import jax.numpy as jnp
from jax import lax
from jax.experimental import pallas as pl
from jax.experimental.pallas import tpu as pltpu
from jax.experimental.pallas import tpu_sc as plsc

N = 1600000
NUM_SEGMENTS = 4096
NW = 32
ROWS_PER_W = N // NW
TILE = 2000
STEPS = ROWS_PER_W // TILE
VECS = TILE // 16

_NEG_INF = float("-inf")


def _sc_stage(t0, t1, t2, vcol, batch_idx, wvec):
    mesh = plsc.VectorSubcoreMesh(core_axis_name="c", subcore_axis_name="s")

    @functools.partial(
        pl.kernel,
        mesh=mesh,
        compiler_params=pltpu.CompilerParams(needs_layout_passes=False),
        out_type=jax.ShapeDtypeStruct((NW, NUM_SEGMENTS), jnp.float32),
        scratch_types=[
            pltpu.VMEM((TILE * 3,), jnp.float32),
            pltpu.VMEM((TILE * 3,), jnp.float32),
            pltpu.VMEM((TILE,), jnp.float32),
            pltpu.VMEM((TILE,), jnp.float32),
            pltpu.VMEM((TILE,), jnp.int32),
            pltpu.VMEM((TILE,), jnp.int32),
            pltpu.VMEM((16, NUM_SEGMENTS), jnp.float32),
            pltpu.VMEM((NUM_SEGMENTS,), jnp.float32),
            pltpu.VMEM((80,), jnp.float32),
            pltpu.SemaphoreType.DMA,
            pltpu.SemaphoreType.DMA,
        ],
    )
    def k(t0_hbm, t1_hbm, t2_hbm, votes_hbm, idx_hbm, wv_hbm, out_hbm,
          tv0, tv1, vv0, vv1, iv0, iv1, acc, red, wv_v, sem0, sem1):
        wid = lax.axis_index("s") * 2 + lax.axis_index("c")
        sems = (sem0, sem1)
        tvs, vvs, ivs = (tv0, tv1), (vv0, vv1), (iv0, iv1)

        lane = lax.iota(jnp.int32, 16)
        ninf = jnp.full((16,), _NEG_INF, jnp.float32)

        handles = [None, None]

        def fire(s):
            b = s % 2
            row0 = wid * ROWS_PER_W + s * TILE
            hs = []
            for j, col in enumerate((t0_hbm, t1_hbm, t2_hbm)):
                hs.append(pltpu.async_copy(
                    col.at[pl.ds(row0, TILE)],
                    tvs[b].at[pl.ds(j * TILE, TILE)], sems[b]))
            hs.append(pltpu.async_copy(
                votes_hbm.at[pl.ds(row0, TILE)], vvs[b], sems[b]))
            hs.append(pltpu.async_copy(
                idx_hbm.at[pl.ds(row0, TILE)], ivs[b], sems[b]))
            handles[b] = hs

        fire(0)

        pltpu.sync_copy(wv_hbm, wv_v)
        w0 = wv_v[pl.ds(0, 16)]
        w1 = wv_v[pl.ds(16, 16)]
        w2 = wv_v[pl.ds(32, 16)]
        w3 = wv_v[pl.ds(48, 16)]
        w4 = wv_v[pl.ds(64, 16)]

        def init_body(j, _):
            for r in range(16):
                acc[r, pl.ds(j * 16, 16)] = ninf
            return 0
        lax.fori_loop(0, NUM_SEGMENTS // 16, init_body, 0)

        for s in range(STEPS):
            if s + 1 < STEPS:
                fire(s + 1)
            b = s % 2
            for h in handles[b]:
                h.wait()

            def vec_body(v, _):
                c0 = tvs[b][pl.ds(v * 16, 16)]
                c1 = tvs[b][pl.ds(TILE + v * 16, 16)]
                c2 = tvs[b][pl.ds(2 * TILE + v * 16, 16)]
                vvv = vvs[b][pl.ds(v * 16, 16)]
                ivv = ivs[b][pl.ds(v * 16, 16)]
                x = c0 * w0 + c1 * w1 + c2 * w2 + vvv * w3 + w4
                g = plsc.load_gather(acc, [lane, ivv])
                plsc.store_scatter(acc, [lane, ivv], jnp.maximum(g, x))
                return 0
            lax.fori_loop(0, VECS, vec_body, 0)

        def red_body(j, _):
            m = acc[0, pl.ds(j * 16, 16)]
            for r in range(1, 16):
                m = jnp.maximum(m, acc[r, pl.ds(j * 16, 16)])
            red[pl.ds(j * 16, 16)] = m
            return 0
        lax.fori_loop(0, NUM_SEGMENTS // 16, red_body, 0)

        pltpu.sync_copy(red, out_hbm.at[wid])

    return k(t0, t1, t2, vcol, batch_idx, wvec)


def _tc_reduce(partial):
    def body(p_ref, o_ref):
        o_ref[...] = jnp.max(p_ref[...], axis=0, keepdims=True)

    return pl.pallas_call(
        body,
        out_shape=jax.ShapeDtypeStruct((1, NUM_SEGMENTS), jnp.float32),
    )(partial)


def kernel(trunk, votes, batch_idx, W, b):
    t0, t1, t2 = trunk[:, 0], trunk[:, 1], trunk[:, 2]
    vcol = votes[:, 0]
    wcat = jnp.concatenate([W[:, 0], b])
    wvec = jnp.repeat(wcat, 16)
    partial = _sc_stage(t0, t1, t2, vcol, batch_idx, wvec)
    out = _tc_reduce(partial)
    return out.reshape(NUM_SEGMENTS, 1)
